# baseline (speedup 1.0000x reference)
import functools

import jax
import jax.numpy as jnp
from jax import lax
from jax.experimental import pallas as pl
from jax.experimental.pallas import tpu as pltpu
from jax.experimental.pallas import tpu_sc as plsc

N, E, D, HID, OUT, G, DC = 10000, 320000, 128, 256, 128, 256, 1024
EPS = 1e-5
NP_ = 10240
NA = 10112
NC, NS = 2, 16
RA = NA // NS
B = 512
NB = NP_ // B
C1 = 160
C2 = 80
IB = 32
GA = G + 16
RW = NP_ // (NC * NS)


def _mesh_kw():
    return dict(mesh=plsc.VectorSubcoreMesh(
        core_axis_name="c", subcore_axis_name="s",
        num_cores=NC, num_subcores=NS))


@functools.lru_cache
def _get_sc_degree():
    @functools.partial(
        pl.kernel,
        out_type=jax.ShapeDtypeStruct((NC, NA, 128), jnp.float32),
        scratch_types=[
            pltpu.VMEM((C2, 128), jnp.int32),
            pltpu.VMEM((128, 128), jnp.float32),
            pltpu.VMEM((128, 128), jnp.float32),
            pltpu.VMEM_SHARED((NA, 128), jnp.float32),
            pltpu.SemaphoreType.DMA,
        ],
        **_mesh_kw(),
    )
    def _sc_degree(dst_hbm, zo_hbm, out_hbm, idxd, zeros_v, ones_v, hist,
                   sem):
        c = lax.axis_index("c")
        s = lax.axis_index("s")
        pltpu.sync_copy(zo_hbm.at[0], zeros_v)
        pltpu.sync_copy(zo_hbm.at[1], ones_v)

        def zcp(j, _):
            pltpu.sync_copy(zeros_v, hist.at[pl.ds(s * RA + j * 128, 128)])
            return 0

        lax.fori_loop(0, 4, zcp, 0)
        pltpu.sync_copy(zeros_v.at[pl.ds(0, 120)],
                        hist.at[pl.ds(s * RA + 512, 120)])
        plsc.subcore_barrier()

        pltpu.sync_copy(dst_hbm.at[c, s], idxd)

        def blk(b, _):
            cps = [
                pltpu.async_copy(ones_v, hist.at[idxd.at[b * 8 + j]],
                                 sem, add=True)
                for j in range(8)
            ]
            for cp in cps:
                cp.wait()
            return 0

        lax.fori_loop(0, C2 // 8, blk, 0)
        plsc.subcore_barrier()
        pltpu.sync_copy(hist.at[pl.ds(s * RA, RA)],
                        out_hbm.at[c, pl.ds(s * RA, RA)])

    return _sc_degree


@functools.lru_cache
def _get_conv_scatter(C, TR, tag, IB=IB, all_table_init=True):
    @functools.partial(
        pl.kernel,
        out_type=jax.ShapeDtypeStruct((NC, NA, 128), jnp.float32),
        scratch_types=[
            pltpu.VMEM((IB, 128), jnp.int32),
            pltpu.VMEM((IB, 128), jnp.int32),
            pltpu.VMEM((128, 128), jnp.float32),
            pltpu.VMEM((128, 128), jnp.float32),
            pltpu.VMEM_SHARED((NA, 128), jnp.float32),
            pltpu.SemaphoreType.DMA,
            pltpu.SemaphoreType.DMA,
        ],
        **_mesh_kw(),
        name=f"conv_scatter_{tag}",
    )
    def _conv(table_hbm, src_hbm, dst_hbm, out_hbm,
              idxs, idxd, rows_a, rows_b, acc, sem_a, sem_b):
        c = lax.axis_index("c")
        s = lax.axis_index("s")

        if all_table_init:
            base = c * NP_ + s * RA
        else:
            base = jnp.where(c == 0, s * RA, TR - 128)

        def icp(j, _):
            pltpu.sync_copy(table_hbm.at[pl.ds(base + j * 128, 128)],
                            acc.at[pl.ds(s * RA + j * 128, 128)])
            return 0

        if not all_table_init:
            def icp(j, _):
                off = jnp.where(c == 0, base + j * 128, base)
                pltpu.sync_copy(table_hbm.at[pl.ds(off, 128)],
                                acc.at[pl.ds(s * RA + j * 128, 128)])
                return 0

        lax.fori_loop(0, 4, icp, 0)
        if all_table_init:
            pltpu.sync_copy(table_hbm.at[pl.ds(base + 512, 120)],
                            acc.at[pl.ds(s * RA + 512, 120)])
        else:
            off5 = jnp.where(c == 0, base + 512, base)
            pltpu.sync_copy(table_hbm.at[pl.ds(off5, 120)],
                            acc.at[pl.ds(s * RA + 512, 120)])
        plsc.subcore_barrier()

        def mblk(m, _):
            pltpu.sync_copy(src_hbm.at[c, s, pl.ds(m * IB, IB)], idxs)
            pltpu.sync_copy(dst_hbm.at[c, s, pl.ds(m * IB, IB)], idxd)
            cp_a0 = pltpu.async_copy(table_hbm.at[idxs.at[0]], rows_a, sem_a)

            def body(i, _):
                t0 = 2 * i
                cp_b = pltpu.async_copy(table_hbm.at[idxs.at[t0 + 1]],
                                        rows_b, sem_b)
                cp_a0.wait()
                pltpu.sync_copy(rows_a, acc.at[idxd.at[t0]], add=True)

                @pl.when(i < IB // 2 - 1)
                def _():
                    pltpu.async_copy(table_hbm.at[idxs.at[t0 + 2]], rows_a,
                                     sem_a)

                cp_b.wait()
                pltpu.sync_copy(rows_b, acc.at[idxd.at[t0 + 1]], add=True)
                return 0

            lax.fori_loop(0, IB // 2, body, 0)
            return 0

        lax.fori_loop(0, C // IB, mblk, 0)
        plsc.subcore_barrier()
        pltpu.sync_copy(acc.at[pl.ds(s * RA, RA)],
                        out_hbm.at[c, pl.ds(s * RA, RA)])

    return _conv


@functools.lru_cache
def _get_segmax():
    @functools.partial(
        pl.kernel,
        out_type=jax.ShapeDtypeStruct((NC * NS, GA, OUT), jnp.float32),
        scratch_types=[
            pltpu.VMEM((RW, OUT), jnp.float32),
            pltpu.VMEM((RW,), jnp.int32),
            pltpu.VMEM((GA, OUT), jnp.float32),
        ],
        **_mesh_kw(),
    )
    def _sc_segmax(w_hbm, ib_hbm, out_hbm, wv, ibv, accv):
        c = lax.axis_index("c")
        s = lax.axis_index("s")
        wid = s * NC + c
        base = wid * RW
        pltpu.sync_copy(w_hbm.at[pl.ds(base, RW)], wv)
        pltpu.sync_copy(ib_hbm.at[pl.ds(base, RW)], ibv)
        neg = jnp.full((16,), -jnp.inf, jnp.float32)

        def ini(k, _):
            accv[k // 8, pl.ds((k % 8) * 16, 16)] = neg
            return 0

        lax.fori_loop(0, GA * 8, ini, 0)

        def grp(q, _):
            ib16 = ibv[pl.ds(q * 16, 16)]
            for k in range(16):
                g = ib16[k]
                r = q * 16 + k
                for j in range(8):
                    sl = pl.ds(j * 16, 16)
                    accv[g, sl] = jnp.maximum(accv[g, sl], wv[r, sl])
            return 0

        lax.fori_loop(0, RW // 16, grp, 0)
        pltpu.sync_copy(accv, out_hbm.at[wid])

    return _sc_segmax


def _t12(x_ref, w1_ref, dc_ref, o_ref, dv_ref):
    h = jnp.dot(x_ref[...], w1_ref[...], preferred_element_type=jnp.float32)
    dc = dc_ref[...]
    dinv = lax.rsqrt(dc[0, :, 0:1] + dc[1, :, 0:1] + 1.0)
    dv_ref[...] = dinv
    o_ref[...] = (h * dinv)[None]


def _t34(s_ref, dv_ref, b_ref, w2_ref, g_ref, be_ref, o_ref,
         zscr, csscr):
    p = pl.program_id(0)

    @pl.when(p < NB)
    def _phase_a():
        i = p
        dinv = dv_ref[...]
        z = jnp.maximum(
            dinv * jnp.concatenate([s_ref[0], s_ref[1]], axis=-1)
            + b_ref[...], 0.0)
        zscr[pl.ds(i * B, B), :] = z
        rows = i * B + lax.broadcasted_iota(jnp.int32, (B, 1), 0)
        zm = jnp.where(rows < N, z, 0.0)
        st = jnp.concatenate(
            [jnp.sum(zm, axis=0, keepdims=True),
             jnp.sum(zm * zm, axis=0, keepdims=True)], axis=0)

        @pl.when(p == 0)
        def _():
            csscr[...] = st

        @pl.when(p > 0)
        def _():
            csscr[...] = csscr[...] + st

    @pl.when(p >= NB)
    def _phase_b():
        i = p - NB
        cs = csscr[...]
        mean = cs[0:1] / N
        var = cs[1:2] / N - mean * mean
        a = lax.rsqrt(var + EPS) * g_ref[...]
        d = be_ref[...] - mean * a
        x = zscr[pl.ds(i * B, B), :] * a + d
        h = jnp.dot(x, w2_ref[...], preferred_element_type=jnp.float32)
        rows = i * B + lax.broadcasted_iota(jnp.int32, (B, 1), 0)
        o_ref[...] = jnp.where(rows < N, h * dv_ref[...], 0.0)


def _bn_affine(cs, gam, bet):
    st = jnp.sum(cs, axis=0)
    mean = st[0:1] / N
    var = st[1:2] / N - mean * mean
    a = lax.rsqrt(var + EPS) * gam
    d = bet - mean * a
    return a, d


def _t5(s_ref, dv_ref, b_ref, w_ref, cs_ref):
    i = pl.program_id(0)
    dinv = dv_ref[...]
    sc = s_ref[0] + s_ref[1]
    w = jnp.maximum(dinv * sc + b_ref[...], 0.0)
    w_ref[...] = w
    rows = i * B + lax.broadcasted_iota(jnp.int32, (B, 1), 0)
    wm = jnp.where(rows < N, w, 0.0)
    st = jnp.concatenate(
        [jnp.sum(wm, axis=0, keepdims=True),
         jnp.sum(wm * wm, axis=0, keepdims=True),
         jnp.zeros((6, w.shape[-1]), jnp.float32)], axis=0)
    cs_ref[...] = st[None]


def _t67(sm_ref, cs_ref, g_ref, be_ref,
         gx_ref, wc1_ref, bc1_ref, gc1_ref, bec1_ref, wc2_ref, bc2_ref,
         od_ref, oc_ref):
    m = sm_ref[0]
    for k in range(1, NC * NS):
        m = jnp.maximum(m, sm_ref[k])
    m = m[:G]
    a, d = _bn_affine(cs_ref[...], g_ref[...], be_ref[...])
    od_ref[...] = m * a + d

    xc = jnp.maximum(
        jnp.dot(gx_ref[...], wc1_ref[...], preferred_element_type=jnp.float32)
        + bc1_ref[...], 0.0)
    mean = jnp.mean(xc, axis=0, keepdims=True)
    var = jnp.mean(xc * xc, axis=0, keepdims=True) - mean * mean
    xn = (xc - mean) * lax.rsqrt(var + EPS) * gc1_ref[...] + bec1_ref[...]
    oc_ref[...] = jnp.maximum(
        jnp.dot(xn, wc2_ref[...], preferred_element_type=jnp.float32)
        + bc2_ref[...], 0.0)


def kernel(drug_feature, drug_adj, ibatch, gexpr_data,
           W1, b1, g1, be1, W2, b2, g2, be2,
           Wc1, bc1, gc1, bec1, Wc2, bc2):
    f32 = jnp.float32
    src = drug_adj[0]
    dst = drug_adj[1]

    pad0 = NC * NS * C2 * 128 - E
    psrc0 = jnp.arange(pad0, dtype=jnp.int32) % N
    pdst0 = N + (jnp.arange(pad0, dtype=jnp.int32) % (NA - N))
    fsrc = jnp.concatenate([src, psrc0])
    fdst = jnp.concatenate([dst, pdst0])
    src_w = fsrc.reshape(NC, NS, C2, 128)
    dst_w = fdst.reshape(NC, NS, C2, 128)

    srcr = fsrc.reshape(1, NS, C1, 128)
    dstr = fdst.reshape(1, NS, C1, 128)
    src1 = jnp.concatenate([srcr, srcr + NP_], axis=0)
    dst1 = jnp.concatenate([dstr, dstr], axis=0)

    x_pad = jnp.concatenate([drug_feature, jnp.zeros((NP_ - N, D), f32)],
                            axis=0)
    ib_pad = jnp.concatenate(
        [ibatch, jnp.full((NP_ - N,), G, jnp.int32)], axis=0)

    zo = jnp.concatenate([jnp.zeros((1, 128, 128), f32),
                          jnp.ones((1, 128, 128), f32)], axis=0)
    degcnt = _get_sc_degree()(dst_w, zo)

    table1, dinv = pl.pallas_call(
        _t12, grid=(NB, NC),
        in_specs=[pl.BlockSpec((B, D), lambda i, c: (i, 0)),
                  pl.BlockSpec((D, D), lambda i, c: (0, c)),
                  pl.BlockSpec((NC, B, 128), lambda i, c: (0, i, 0))],
        out_specs=[pl.BlockSpec((1, B, D), lambda i, c: (c, i, 0)),
                   pl.BlockSpec((B, 1), lambda i, c: (i, 0))],
        out_shape=[jax.ShapeDtypeStruct((NC, NP_, D), f32),
                   jax.ShapeDtypeStruct((NP_, 1), f32)],
    )(x_pad, W1, degcnt)

    scat1 = _get_conv_scatter(C1, NC * NP_, "c1")(
        table1.reshape(NC * NP_, D), src1, dst1)

    table2 = pl.pallas_call(
        _t34, grid=(2 * NB,),
        in_specs=[
            pl.BlockSpec((NC, B, D),
                         lambda p: (0, jnp.minimum(p, NB - 1), 0)),
            pl.BlockSpec((B, 1),
                         lambda p: (jnp.where(p < NB, p, p - NB), 0)),
            pl.BlockSpec((1, HID), lambda p: (0, 0)),
            pl.BlockSpec((HID, OUT), lambda p: (0, 0)),
            pl.BlockSpec((1, HID), lambda p: (0, 0)),
            pl.BlockSpec((1, HID), lambda p: (0, 0))],
        out_specs=pl.BlockSpec((B, OUT),
                               lambda p: (jnp.where(p < NB, 0, p - NB), 0)),
        out_shape=jax.ShapeDtypeStruct((NP_, OUT), f32),
        scratch_shapes=[pltpu.VMEM((NP_, HID), f32),
                        pltpu.VMEM((2, HID), f32)],
    )(scat1, dinv, b1.reshape(1, HID), W2,
      g1.reshape(1, HID), be1.reshape(1, HID))

    scat2 = _get_conv_scatter(C2, NP_, "c2", 40, False)(table2, src_w,
                                                        dst_w)

    w2d, cs2 = pl.pallas_call(
        _t5, grid=(NB,),
        in_specs=[pl.BlockSpec((NC, B, OUT), lambda i: (0, i, 0)),
                  pl.BlockSpec((B, 1), lambda i: (i, 0)),
                  pl.BlockSpec((1, OUT), lambda i: (0, 0))],
        out_specs=[pl.BlockSpec((B, OUT), lambda i: (i, 0)),
                   pl.BlockSpec((1, 8, OUT), lambda i: (i, 0, 0))],
        out_shape=[jax.ShapeDtypeStruct((NP_, OUT), f32),
                   jax.ShapeDtypeStruct((NB, 8, OUT), f32)],
    )(scat2, dinv, b2.reshape(1, OUT))

    smp = _get_segmax()(w2d, ib_pad)

    x_drug, x_cell = pl.pallas_call(
        _t67, grid=(1,),
        in_specs=[pl.BlockSpec((NC * NS, GA, OUT), lambda i: (0, 0, 0)),
                  pl.BlockSpec((NB, 8, OUT), lambda i: (0, 0, 0)),
                  pl.BlockSpec((1, OUT), lambda i: (0, 0)),
                  pl.BlockSpec((1, OUT), lambda i: (0, 0)),
                  pl.BlockSpec((G, DC), lambda i: (0, 0)),
                  pl.BlockSpec((DC, 256), lambda i: (0, 0)),
                  pl.BlockSpec((1, 256), lambda i: (0, 0)),
                  pl.BlockSpec((1, 256), lambda i: (0, 0)),
                  pl.BlockSpec((1, 256), lambda i: (0, 0)),
                  pl.BlockSpec((256, OUT), lambda i: (0, 0)),
                  pl.BlockSpec((1, OUT), lambda i: (0, 0))],
        out_specs=[pl.BlockSpec((G, OUT), lambda i: (0, 0)),
                   pl.BlockSpec((G, OUT), lambda i: (0, 0))],
        out_shape=[jax.ShapeDtypeStruct((G, OUT), f32),
                   jax.ShapeDtypeStruct((G, OUT), f32)],
    )(smp, cs2, g2.reshape(1, OUT), be2.reshape(1, OUT),
      gexpr_data, Wc1, bc1.reshape(1, 256), gc1.reshape(1, 256),
      bec1.reshape(1, 256), Wc2, bc2.reshape(1, OUT))

    return (x_drug, x_cell)


# speedup vs baseline: 1.0166x; 1.0166x over previous
import functools

import jax
import jax.numpy as jnp
from jax import lax
from jax.experimental import pallas as pl
from jax.experimental.pallas import tpu as pltpu
from jax.experimental.pallas import tpu_sc as plsc

N, E, D, HID, OUT, G, DC = 10000, 320000, 128, 256, 128, 256, 1024
EPS = 1e-5
NP_ = 10240
NA = 10112
NC, NS = 2, 16
RA = NA // NS
B = 512
NB = NP_ // B
C1 = 160
C2 = 80
IB = 32
GA = G + 16
RW = NP_ // (NC * NS)


def _mesh_kw():
    return dict(mesh=plsc.VectorSubcoreMesh(
        core_axis_name="c", subcore_axis_name="s",
        num_cores=NC, num_subcores=NS))


@functools.lru_cache
def _get_sc_degree():
    ND = 10240
    RD = ND // NS

    @functools.partial(
        pl.kernel,
        out_type=jax.ShapeDtypeStruct((NC, ND), jnp.float32),
        scratch_types=[
            pltpu.VMEM((C2, 128), jnp.int32),
            pltpu.VMEM((128, 128), jnp.float32),
            pltpu.VMEM((128, 128), jnp.float32),
            pltpu.VMEM((RD,), jnp.float32),
            pltpu.VMEM_SHARED((ND, 128), jnp.float32),
            pltpu.SemaphoreType.DMA,
        ],
        **_mesh_kw(),
    )
    def _sc_degree(dst_hbm, zo_hbm, out_hbm, idxd, zeros_v, ones_v, cmp_v,
                   hist, sem):
        c = lax.axis_index("c")
        s = lax.axis_index("s")
        pltpu.sync_copy(zo_hbm.at[0], zeros_v)
        pltpu.sync_copy(zo_hbm.at[1], ones_v)

        def zcp(j, _):
            pltpu.sync_copy(zeros_v, hist.at[pl.ds(s * RD + j * 128, 128)])
            return 0

        lax.fori_loop(0, RD // 128, zcp, 0)
        plsc.subcore_barrier()

        pltpu.sync_copy(dst_hbm.at[c, s], idxd)

        def blk(b, _):
            cps = [
                pltpu.async_copy(ones_v, hist.at[idxd.at[b * 8 + j]],
                                 sem, add=True)
                for j in range(8)
            ]
            for cp in cps:
                cp.wait()
            return 0

        lax.fori_loop(0, C2 // 8, blk, 0)
        plsc.subcore_barrier()

        lanes = lax.iota(jnp.int32, 16)

        def tile(t, _):
            pltpu.sync_copy(hist.at[pl.ds(s * RD + t * 128, 128)], zeros_v)

            def grp(g, _):
                acc16 = jnp.zeros((16,), jnp.float32)
                for k in range(16):
                    v = zeros_v[g * 16 + k, pl.ds(0, 16)]
                    acc16 = jnp.where(lanes == k, v[0], acc16)
                cmp_v[pl.ds(t * 128 + g * 16, 16)] = acc16
                return 0

            lax.fori_loop(0, 8, grp, 0)
            return 0

        lax.fori_loop(0, RD // 128, tile, 0)
        pltpu.sync_copy(cmp_v, out_hbm.at[c, pl.ds(s * RD, RD)])

    return _sc_degree


@functools.lru_cache
def _get_conv_scatter(C, TR, tag, IB=IB, all_table_init=True):
    @functools.partial(
        pl.kernel,
        out_type=jax.ShapeDtypeStruct((NC, NA, 128), jnp.float32),
        scratch_types=[
            pltpu.VMEM((IB, 128), jnp.int32),
            pltpu.VMEM((IB, 128), jnp.int32),
            pltpu.VMEM((128, 128), jnp.float32),
            pltpu.VMEM((128, 128), jnp.float32),
            pltpu.VMEM_SHARED((NA, 128), jnp.float32),
            pltpu.SemaphoreType.DMA,
            pltpu.SemaphoreType.DMA,
        ],
        **_mesh_kw(),
        name=f"conv_scatter_{tag}",
    )
    def _conv(table_hbm, src_hbm, dst_hbm, out_hbm,
              idxs, idxd, rows_a, rows_b, acc, sem_a, sem_b):
        c = lax.axis_index("c")
        s = lax.axis_index("s")

        if all_table_init:
            base = c * NP_ + s * RA
        else:
            base = jnp.where(c == 0, s * RA, TR - 128)

        def icp(j, _):
            pltpu.sync_copy(table_hbm.at[pl.ds(base + j * 128, 128)],
                            acc.at[pl.ds(s * RA + j * 128, 128)])
            return 0

        if not all_table_init:
            def icp(j, _):
                off = jnp.where(c == 0, base + j * 128, base)
                pltpu.sync_copy(table_hbm.at[pl.ds(off, 128)],
                                acc.at[pl.ds(s * RA + j * 128, 128)])
                return 0

        lax.fori_loop(0, 4, icp, 0)
        if all_table_init:
            pltpu.sync_copy(table_hbm.at[pl.ds(base + 512, 120)],
                            acc.at[pl.ds(s * RA + 512, 120)])
        else:
            off5 = jnp.where(c == 0, base + 512, base)
            pltpu.sync_copy(table_hbm.at[pl.ds(off5, 120)],
                            acc.at[pl.ds(s * RA + 512, 120)])
        plsc.subcore_barrier()

        def mblk(m, _):
            pltpu.sync_copy(src_hbm.at[c, s, pl.ds(m * IB, IB)], idxs)
            pltpu.sync_copy(dst_hbm.at[c, s, pl.ds(m * IB, IB)], idxd)
            cp_a0 = pltpu.async_copy(table_hbm.at[idxs.at[0]], rows_a, sem_a)

            def body(i, _):
                t0 = 2 * i
                cp_b = pltpu.async_copy(table_hbm.at[idxs.at[t0 + 1]],
                                        rows_b, sem_b)
                cp_a0.wait()
                pltpu.sync_copy(rows_a, acc.at[idxd.at[t0]], add=True)

                @pl.when(i < IB // 2 - 1)
                def _():
                    pltpu.async_copy(table_hbm.at[idxs.at[t0 + 2]], rows_a,
                                     sem_a)

                cp_b.wait()
                pltpu.sync_copy(rows_b, acc.at[idxd.at[t0 + 1]], add=True)
                return 0

            lax.fori_loop(0, IB // 2, body, 0)
            return 0

        lax.fori_loop(0, C // IB, mblk, 0)
        plsc.subcore_barrier()
        pltpu.sync_copy(acc.at[pl.ds(s * RA, RA)],
                        out_hbm.at[c, pl.ds(s * RA, RA)])

    return _conv


@functools.lru_cache
def _get_segmax():
    @functools.partial(
        pl.kernel,
        out_type=jax.ShapeDtypeStruct((NC * NS, GA, OUT), jnp.float32),
        scratch_types=[
            pltpu.VMEM((RW, OUT), jnp.float32),
            pltpu.VMEM((RW,), jnp.int32),
            pltpu.VMEM((GA, OUT), jnp.float32),
        ],
        **_mesh_kw(),
    )
    def _sc_segmax(w_hbm, ib_hbm, out_hbm, wv, ibv, accv):
        c = lax.axis_index("c")
        s = lax.axis_index("s")
        wid = s * NC + c
        base = wid * RW
        pltpu.sync_copy(w_hbm.at[pl.ds(base, RW)], wv)
        pltpu.sync_copy(ib_hbm.at[pl.ds(base, RW)], ibv)
        neg = jnp.full((16,), -jnp.inf, jnp.float32)

        def ini(k, _):
            accv[k // 8, pl.ds((k % 8) * 16, 16)] = neg
            return 0

        lax.fori_loop(0, GA * 8, ini, 0)

        def grp(q, _):
            ib16 = ibv[pl.ds(q * 16, 16)]
            for k in range(16):
                g = ib16[k]
                r = q * 16 + k
                for j in range(8):
                    sl = pl.ds(j * 16, 16)
                    accv[g, sl] = jnp.maximum(accv[g, sl], wv[r, sl])
            return 0

        lax.fori_loop(0, RW // 16, grp, 0)
        pltpu.sync_copy(accv, out_hbm.at[wid])

    return _sc_segmax


def _t12(x_ref, w1_ref, dc_ref, o_ref, dv_ref):
    h = jnp.dot(x_ref[...], w1_ref[...], preferred_element_type=jnp.float32)
    dc = dc_ref[...]
    deg = dc[0:1] + dc[1:2] + 1.0
    dinv = jnp.transpose(lax.rsqrt(deg), (1, 0))
    dv_ref[...] = dinv
    o_ref[...] = (h * dinv)[None]


def _t34(s_ref, dv_ref, b_ref, w2_ref, g_ref, be_ref, o_ref,
         zscr, csscr):
    p = pl.program_id(0)

    @pl.when(p < NB)
    def _phase_a():
        i = p
        dinv = dv_ref[...]
        z = jnp.maximum(
            dinv * jnp.concatenate([s_ref[0], s_ref[1]], axis=-1)
            + b_ref[...], 0.0)
        zscr[pl.ds(i * B, B), :] = z
        rows = i * B + lax.broadcasted_iota(jnp.int32, (B, 1), 0)
        zm = jnp.where(rows < N, z, 0.0)
        st = jnp.concatenate(
            [jnp.sum(zm, axis=0, keepdims=True),
             jnp.sum(zm * zm, axis=0, keepdims=True)], axis=0)

        @pl.when(p == 0)
        def _():
            csscr[...] = st

        @pl.when(p > 0)
        def _():
            csscr[...] = csscr[...] + st

    @pl.when(p >= NB)
    def _phase_b():
        i = p - NB
        cs = csscr[...]
        mean = cs[0:1] / N
        var = cs[1:2] / N - mean * mean
        a = lax.rsqrt(var + EPS) * g_ref[...]
        d = be_ref[...] - mean * a
        x = zscr[pl.ds(i * B, B), :] * a + d
        h = jnp.dot(x, w2_ref[...], preferred_element_type=jnp.float32)
        rows = i * B + lax.broadcasted_iota(jnp.int32, (B, 1), 0)
        o_ref[...] = jnp.where(rows < N, h * dv_ref[...], 0.0)


def _bn_affine(cs, gam, bet):
    st = jnp.sum(cs, axis=0)
    mean = st[0:1] / N
    var = st[1:2] / N - mean * mean
    a = lax.rsqrt(var + EPS) * gam
    d = bet - mean * a
    return a, d


def _t5(s_ref, dv_ref, b_ref, w_ref, cs_ref):
    i = pl.program_id(0)
    dinv = dv_ref[...]
    sc = s_ref[0] + s_ref[1]
    w = jnp.maximum(dinv * sc + b_ref[...], 0.0)
    w_ref[...] = w
    rows = i * B + lax.broadcasted_iota(jnp.int32, (B, 1), 0)
    wm = jnp.where(rows < N, w, 0.0)
    st = jnp.concatenate(
        [jnp.sum(wm, axis=0, keepdims=True),
         jnp.sum(wm * wm, axis=0, keepdims=True),
         jnp.zeros((6, w.shape[-1]), jnp.float32)], axis=0)
    cs_ref[...] = st[None]


def _t67(sm_ref, cs_ref, g_ref, be_ref,
         gx_ref, wc1_ref, bc1_ref, gc1_ref, bec1_ref, wc2_ref, bc2_ref,
         od_ref, oc_ref):
    m = sm_ref[0]
    for k in range(1, NC * NS):
        m = jnp.maximum(m, sm_ref[k])
    m = m[:G]
    a, d = _bn_affine(cs_ref[...], g_ref[...], be_ref[...])
    od_ref[...] = m * a + d

    xc = jnp.maximum(
        jnp.dot(gx_ref[...], wc1_ref[...], preferred_element_type=jnp.float32)
        + bc1_ref[...], 0.0)
    mean = jnp.mean(xc, axis=0, keepdims=True)
    var = jnp.mean(xc * xc, axis=0, keepdims=True) - mean * mean
    xn = (xc - mean) * lax.rsqrt(var + EPS) * gc1_ref[...] + bec1_ref[...]
    oc_ref[...] = jnp.maximum(
        jnp.dot(xn, wc2_ref[...], preferred_element_type=jnp.float32)
        + bc2_ref[...], 0.0)


def kernel(drug_feature, drug_adj, ibatch, gexpr_data,
           W1, b1, g1, be1, W2, b2, g2, be2,
           Wc1, bc1, gc1, bec1, Wc2, bc2):
    f32 = jnp.float32
    src = drug_adj[0]
    dst = drug_adj[1]

    pad0 = NC * NS * C2 * 128 - E
    psrc0 = jnp.arange(pad0, dtype=jnp.int32) % N
    pdst0 = N + (jnp.arange(pad0, dtype=jnp.int32) % (NA - N))
    fsrc = jnp.concatenate([src, psrc0])
    fdst = jnp.concatenate([dst, pdst0])
    src_w = fsrc.reshape(NC, NS, C2, 128)
    dst_w = fdst.reshape(NC, NS, C2, 128)

    srcr = fsrc.reshape(1, NS, C1, 128)
    dstr = fdst.reshape(1, NS, C1, 128)
    src1 = jnp.concatenate([srcr, srcr + NP_], axis=0)
    dst1 = jnp.concatenate([dstr, dstr], axis=0)

    x_pad = jnp.concatenate([drug_feature, jnp.zeros((NP_ - N, D), f32)],
                            axis=0)
    ib_pad = jnp.concatenate(
        [ibatch, jnp.full((NP_ - N,), G, jnp.int32)], axis=0)

    zo = jnp.concatenate([jnp.zeros((1, 128, 128), f32),
                          jnp.ones((1, 128, 128), f32)], axis=0)
    degcnt = _get_sc_degree()(dst_w, zo)

    table1, dinv = pl.pallas_call(
        _t12, grid=(NB, NC),
        in_specs=[pl.BlockSpec((B, D), lambda i, c: (i, 0)),
                  pl.BlockSpec((D, D), lambda i, c: (0, c)),
                  pl.BlockSpec((NC, B), lambda i, c: (0, i))],
        out_specs=[pl.BlockSpec((1, B, D), lambda i, c: (c, i, 0)),
                   pl.BlockSpec((B, 1), lambda i, c: (i, 0))],
        out_shape=[jax.ShapeDtypeStruct((NC, NP_, D), f32),
                   jax.ShapeDtypeStruct((NP_, 1), f32)],
    )(x_pad, W1, degcnt)

    scat1 = _get_conv_scatter(C1, NC * NP_, "c1")(
        table1.reshape(NC * NP_, D), src1, dst1)

    table2 = pl.pallas_call(
        _t34, grid=(2 * NB,),
        in_specs=[
            pl.BlockSpec((NC, B, D),
                         lambda p: (0, jnp.minimum(p, NB - 1), 0)),
            pl.BlockSpec((B, 1),
                         lambda p: (jnp.where(p < NB, p, p - NB), 0)),
            pl.BlockSpec((1, HID), lambda p: (0, 0)),
            pl.BlockSpec((HID, OUT), lambda p: (0, 0)),
            pl.BlockSpec((1, HID), lambda p: (0, 0)),
            pl.BlockSpec((1, HID), lambda p: (0, 0))],
        out_specs=pl.BlockSpec((B, OUT),
                               lambda p: (jnp.where(p < NB, 0, p - NB), 0)),
        out_shape=jax.ShapeDtypeStruct((NP_, OUT), f32),
        scratch_shapes=[pltpu.VMEM((NP_, HID), f32),
                        pltpu.VMEM((2, HID), f32)],
    )(scat1, dinv, b1.reshape(1, HID), W2,
      g1.reshape(1, HID), be1.reshape(1, HID))

    scat2 = _get_conv_scatter(C2, NP_, "c2", 40, False)(table2, src_w,
                                                        dst_w)

    w2d, cs2 = pl.pallas_call(
        _t5, grid=(NB,),
        in_specs=[pl.BlockSpec((NC, B, OUT), lambda i: (0, i, 0)),
                  pl.BlockSpec((B, 1), lambda i: (i, 0)),
                  pl.BlockSpec((1, OUT), lambda i: (0, 0))],
        out_specs=[pl.BlockSpec((B, OUT), lambda i: (i, 0)),
                   pl.BlockSpec((1, 8, OUT), lambda i: (i, 0, 0))],
        out_shape=[jax.ShapeDtypeStruct((NP_, OUT), f32),
                   jax.ShapeDtypeStruct((NB, 8, OUT), f32)],
    )(scat2, dinv, b2.reshape(1, OUT))

    smp = _get_segmax()(w2d, ib_pad)

    x_drug, x_cell = pl.pallas_call(
        _t67, grid=(1,),
        in_specs=[pl.BlockSpec((NC * NS, GA, OUT), lambda i: (0, 0, 0)),
                  pl.BlockSpec((NB, 8, OUT), lambda i: (0, 0, 0)),
                  pl.BlockSpec((1, OUT), lambda i: (0, 0)),
                  pl.BlockSpec((1, OUT), lambda i: (0, 0)),
                  pl.BlockSpec((G, DC), lambda i: (0, 0)),
                  pl.BlockSpec((DC, 256), lambda i: (0, 0)),
                  pl.BlockSpec((1, 256), lambda i: (0, 0)),
                  pl.BlockSpec((1, 256), lambda i: (0, 0)),
                  pl.BlockSpec((1, 256), lambda i: (0, 0)),
                  pl.BlockSpec((256, OUT), lambda i: (0, 0)),
                  pl.BlockSpec((1, OUT), lambda i: (0, 0))],
        out_specs=[pl.BlockSpec((G, OUT), lambda i: (0, 0)),
                   pl.BlockSpec((G, OUT), lambda i: (0, 0))],
        out_shape=[jax.ShapeDtypeStruct((G, OUT), f32),
                   jax.ShapeDtypeStruct((G, OUT), f32)],
    )(smp, cs2, g2.reshape(1, OUT), be2.reshape(1, OUT),
      gexpr_data, Wc1, bc1.reshape(1, 256), gc1.reshape(1, 256),
      bec1.reshape(1, 256), Wc2, bc2.reshape(1, OUT))

    return (x_drug, x_cell)
